# baseline (speedup 1.0000x reference)
import functools

import jax
import jax.numpy as jnp
from jax import lax
from jax.experimental import pallas as pl
from jax.experimental.pallas import tpu as pltpu
from jax.experimental.pallas import tpu_sc as plsc

B = 4096
H = 64
K = 100
NW = 32
BPW = B // NW
NT = 1000000
HALF = 1 << 19
BL = 4096


def _tc_build_pairs_body(x0_ref, x1_ref, out_ref):
    out_ref[...] = jnp.concatenate([x0_ref[...].T, x1_ref[...].T], axis=1)


def _tc_build_pairs(table_t):
    return pl.pallas_call(
        _tc_build_pairs_body,
        grid=(HALF // BL,),
        in_specs=[
            pl.BlockSpec((H, BL), lambda j: (0, j)),
            pl.BlockSpec(
                (H, BL),
                lambda j: (0, jnp.minimum(j + HALF // BL,
                                          (NT + BL - 1) // BL - 1))),
        ],
        out_specs=pl.BlockSpec((BL, 2 * H), lambda j: (j, 0)),
        out_shape=jax.ShapeDtypeStruct((HALF, 2 * H), jnp.float32),
    )(table_t, table_t)


def _sc_mesh():
    return plsc.VectorSubcoreMesh(core_axis_name="c", subcore_axis_name="s")


def _sc_gather_items(pair_t, t2i):
    mesh = _sc_mesh()
    nc = mesh.num_cores

    @functools.partial(
        pl.kernel,
        out_type=jax.ShapeDtypeStruct((K, B, 128), jnp.float32),
        mesh=mesh,
        scratch_types=[
            pltpu.VMEM((K, BPW), jnp.int32),
            pltpu.VMEM((BPW, 128), jnp.float32),
            pltpu.VMEM((BPW, 128), jnp.float32),
            pltpu.SemaphoreType.DMA,
            pltpu.SemaphoreType.DMA,
        ],
    )
    def k(pair_hbm, t2i_hbm, pairs_out, pv, ra, rb, sema, semb):
        wid = lax.axis_index("s") * nc + lax.axis_index("c")
        b0 = wid * BPW

        pltpu.sync_copy(pair_hbm.at[:, pl.ds(b0, BPW)], pv)

        def gather(k_slot, buf, sem):
            pltpu.make_async_copy(t2i_hbm.at[pv.at[k_slot]], buf, sem).start()

        def flush(k_slot, buf, sem):
            pltpu.make_async_copy(t2i_hbm.at[pv.at[k_slot]], buf, sem).wait()
            pltpu.sync_copy(buf, pairs_out.at[k_slot, pl.ds(b0, BPW), :])

        gather(0, ra, sema)

        def body(j, _):
            k0 = 2 * j
            gather(k0 + 1, rb, semb)
            flush(k0, ra, sema)

            @pl.when(k0 + 2 < K)
            def _():
                gather(k0 + 2, ra, sema)

            flush(k0 + 1, rb, semb)
            return 0

        lax.fori_loop(0, K // 2, body, 0)

    return k(pair_t, t2i)


def _tc_user_cols_body(users_ref, utab_ref, out_ref, ta, tb, sema, semb):
    j = pl.program_id(0)
    lane = jax.lax.broadcasted_iota(jnp.int32, (H, 128), 1)

    def cp(i, buf, sem):
        u = users_ref[0, j * 128 + i]
        base = pl.multiple_of((u >> 7) * 128, 128)
        return pltpu.make_async_copy(
            utab_ref.at[:, pl.ds(base, 128)], buf, sem)

    bufs = (ta, tb)
    sems = (sema, semb)
    cp(0, ta, sema).start()
    acc = jnp.zeros((H, 128), jnp.float32)
    for i in range(128):
        if i + 1 < 128:
            cp(i + 1, bufs[(i + 1) % 2], sems[(i + 1) % 2]).start()
        cp(i, bufs[i % 2], sems[i % 2]).wait()
        u = users_ref[0, j * 128 + i]
        rolled = pltpu.roll(bufs[i % 2][...], i - (u & 127), 1)
        acc = jnp.where(lane == i, rolled, acc)
    out_ref[...] = acc


def _tc_user_cols(users2d, table_t):
    return pl.pallas_call(
        _tc_user_cols_body,
        grid=(B // 128,),
        in_specs=[
            pl.BlockSpec(memory_space=pltpu.SMEM),
            pl.BlockSpec(memory_space=pl.ANY),
        ],
        out_specs=pl.BlockSpec((H, 128), lambda j: (0, j)),
        out_shape=jax.ShapeDtypeStruct((H, B), jnp.float32),
        scratch_shapes=[
            pltpu.VMEM((H, 128), jnp.float32),
            pltpu.VMEM((H, 128), jnp.float32),
            pltpu.SemaphoreType.DMA,
            pltpu.SemaphoreType.DMA,
        ],
    )(users2d, table_t)


def _tc_finish_body(idx_ref, pairs_ref, ue_ref, allt_ref, rat_ref):
    ue = ue_ref[...]
    x = pairs_ref[...]
    xt = jnp.transpose(x, (0, 2, 1))
    par = (idx_ref[...] >= HALF)[:, None, :]
    sel = jnp.where(par, xt[:, H:, :], xt[:, :H, :])
    allt_ref[...] = sel
    rat_ref[...] = jnp.sum(sel * ue[None], axis=1)


def _tc_finish(idx_t, pairs, ue_t):
    bb = 128
    return pl.pallas_call(
        _tc_finish_body,
        grid=(B // bb,),
        in_specs=[
            pl.BlockSpec((K, bb), lambda i: (0, i)),
            pl.BlockSpec((K, bb, 128), lambda i: (0, i, 0)),
            pl.BlockSpec((H, bb), lambda i: (0, i)),
        ],
        out_specs=[
            pl.BlockSpec((K, H, bb), lambda i: (0, 0, i)),
            pl.BlockSpec((K, bb), lambda i: (0, i)),
        ],
        out_shape=[
            jax.ShapeDtypeStruct((K, H, B), jnp.float32),
            jax.ShapeDtypeStruct((K, B), jnp.float32),
        ],
    )(idx_t, pairs, ue_t)


def kernel(users, pos_items, neg_items, user_table, item_table):
    users = users.astype(jnp.int32)
    pos_items = pos_items.astype(jnp.int32)
    item_idx = jnp.concatenate(
        [pos_items[:, None], neg_items.astype(jnp.int32)], axis=1)
    idx_t = item_idx.T
    pair_t = idx_t & (HALF - 1)

    t2i = _tc_build_pairs(item_table.T)
    pairs = _sc_gather_items(pair_t, t2i)
    ue_t = _tc_user_cols(users[None, :], user_table.T)
    all_t, rating_t = _tc_finish(idx_t, pairs, ue_t)
    pe_t = all_t[0]

    return (ue_t.T, pe_t.T, all_t.transpose(2, 0, 1), rating_t.T)


# speedup vs baseline: 3.1713x; 3.1713x over previous
import functools

import jax
import jax.numpy as jnp
from jax import lax
from jax.experimental import pallas as pl
from jax.experimental.pallas import tpu as pltpu
from jax.experimental.pallas import tpu_sc as plsc

B = 4096
H = 64
K = 100
NW = 32
BPW = B // NW
NT = 1000000
HALF = 1 << 19
BL = 8192


def _tc_build_pairs_body(x0_ref, x1_ref, out_ref):
    out_ref[...] = jnp.concatenate([x0_ref[...].T, x1_ref[...].T], axis=1)


def _tc_build_pairs(table_t):
    return pl.pallas_call(
        _tc_build_pairs_body,
        grid=(HALF // BL,),
        in_specs=[
            pl.BlockSpec((H, BL), lambda j: (0, j)),
            pl.BlockSpec(
                (H, BL),
                lambda j: (0, jnp.minimum(j + HALF // BL,
                                          (NT + BL - 1) // BL - 1))),
        ],
        out_specs=pl.BlockSpec((BL, 2 * H), lambda j: (j, 0)),
        out_shape=jax.ShapeDtypeStruct((HALF, 2 * H), jnp.float32),
    )(table_t, table_t)


def _sc_mesh():
    return plsc.VectorSubcoreMesh(core_axis_name="c", subcore_axis_name="s")


def _sc_gather_items(pair_t, t2i):
    mesh = _sc_mesh()
    nc = mesh.num_cores

    @functools.partial(
        pl.kernel,
        out_type=jax.ShapeDtypeStruct((K, B, 128), jnp.float32),
        mesh=mesh,
        scratch_types=[
            pltpu.VMEM((K, BPW), jnp.int32),
            pltpu.VMEM((BPW, 128), jnp.float32),
            pltpu.VMEM((BPW, 128), jnp.float32),
            pltpu.SemaphoreType.DMA,
            pltpu.SemaphoreType.DMA,
        ],
    )
    def k(pair_hbm, t2i_hbm, pairs_out, pv, ra, rb, sema, semb):
        wid = lax.axis_index("s") * nc + lax.axis_index("c")
        b0 = wid * BPW

        pltpu.sync_copy(pair_hbm.at[:, pl.ds(b0, BPW)], pv)

        def gather(k_slot, buf, sem):
            pltpu.make_async_copy(t2i_hbm.at[pv.at[k_slot]], buf, sem).start()

        def flush(k_slot, buf, sem):
            pltpu.make_async_copy(t2i_hbm.at[pv.at[k_slot]], buf, sem).wait()
            pltpu.sync_copy(buf, pairs_out.at[k_slot, pl.ds(b0, BPW), :])

        gather(0, ra, sema)

        def body(j, _):
            k0 = 2 * j
            gather(k0 + 1, rb, semb)
            flush(k0, ra, sema)

            @pl.when(k0 + 2 < K)
            def _():
                gather(k0 + 2, ra, sema)

            flush(k0 + 1, rb, semb)
            return 0

        lax.fori_loop(0, K // 2, body, 0)

    return k(pair_t, t2i)


def _sc_gather_users(upair, t2u):
    mesh = _sc_mesh()
    nc = mesh.num_cores

    @functools.partial(
        pl.kernel,
        out_type=jax.ShapeDtypeStruct((B, 128), jnp.float32),
        mesh=mesh,
        scratch_types=[
            pltpu.VMEM((BPW,), jnp.int32),
            pltpu.VMEM((BPW, 128), jnp.float32),
            pltpu.SemaphoreType.DMA,
        ],
    )
    def k(upair_hbm, t2u_hbm, upairs_out, upv, ra, sema):
        wid = lax.axis_index("s") * nc + lax.axis_index("c")
        b0 = wid * BPW
        pltpu.sync_copy(upair_hbm.at[pl.ds(b0, BPW)], upv)
        pltpu.make_async_copy(t2u_hbm.at[upv], ra, sema).start()
        pltpu.make_async_copy(t2u_hbm.at[upv], ra, sema).wait()
        pltpu.sync_copy(ra, upairs_out.at[pl.ds(b0, BPW)])

    return k(upair, t2u)


def _tc_finish_body(idx_ref, users_ref, pairs_ref, upairs_ref,
                    uet_ref, allt_ref, rat_ref):
    up = upairs_ref[...]
    upt = up.T
    upar = users_ref[...] >= HALF
    ue = jnp.where(upar, upt[H:], upt[:H])
    uet_ref[...] = ue

    x = pairs_ref[...]
    xt = jnp.transpose(x, (0, 2, 1))
    par = (idx_ref[...] >= HALF)[:, None, :]
    sel = jnp.where(par, xt[:, H:, :], xt[:, :H, :])
    allt_ref[...] = sel
    rat_ref[...] = jnp.sum(sel * ue[None], axis=1)


def _tc_finish(idx_t, users2d, pairs, upairs):
    bb = 256
    return pl.pallas_call(
        _tc_finish_body,
        grid=(B // bb,),
        in_specs=[
            pl.BlockSpec((K, bb), lambda i: (0, i)),
            pl.BlockSpec((1, bb), lambda i: (0, i)),
            pl.BlockSpec((K, bb, 128), lambda i: (0, i, 0)),
            pl.BlockSpec((bb, 128), lambda i: (i, 0)),
        ],
        out_specs=[
            pl.BlockSpec((H, bb), lambda i: (0, i)),
            pl.BlockSpec((K, H, bb), lambda i: (0, 0, i)),
            pl.BlockSpec((K, bb), lambda i: (0, i)),
        ],
        out_shape=[
            jax.ShapeDtypeStruct((H, B), jnp.float32),
            jax.ShapeDtypeStruct((K, H, B), jnp.float32),
            jax.ShapeDtypeStruct((K, B), jnp.float32),
        ],
    )(idx_t, users2d, pairs, upairs)


def kernel(users, pos_items, neg_items, user_table, item_table):
    users = users.astype(jnp.int32)
    pos_items = pos_items.astype(jnp.int32)
    item_idx = jnp.concatenate(
        [pos_items[:, None], neg_items.astype(jnp.int32)], axis=1)
    idx_t = item_idx.T
    pair_t = idx_t & (HALF - 1)
    upair = users & (HALF - 1)

    t2i = _tc_build_pairs(item_table.T)
    t2u = _tc_build_pairs(user_table.T)

    pairs = _sc_gather_items(pair_t, t2i)
    upairs = _sc_gather_users(upair, t2u)
    ue_t, all_t, rating_t = _tc_finish(idx_t, users[None, :], pairs, upairs)
    pe_t = all_t[0]

    return (ue_t.T, pe_t.T, all_t.transpose(2, 0, 1), rating_t.T)


# speedup vs baseline: 3.2681x; 1.0305x over previous
import functools

import jax
import jax.numpy as jnp
from jax import lax
from jax.experimental import pallas as pl
from jax.experimental.pallas import tpu as pltpu
from jax.experimental.pallas import tpu_sc as plsc

B = 4096
H = 64
K = 100
NW = 32
BPW = B // NW
NT = 1000000
HALF = 1 << 19
BL = 16384


def _tc_build_pairs_body(x0_ref, x1_ref, out_ref):
    out_ref[...] = jnp.concatenate([x0_ref[...].T, x1_ref[...].T], axis=1)


def _tc_build_pairs(table_t):
    return pl.pallas_call(
        _tc_build_pairs_body,
        grid=(HALF // BL,),
        in_specs=[
            pl.BlockSpec((H, BL), lambda j: (0, j)),
            pl.BlockSpec(
                (H, BL),
                lambda j: (0, jnp.minimum(j + HALF // BL,
                                          (NT + BL - 1) // BL - 1))),
        ],
        out_specs=pl.BlockSpec((BL, 2 * H), lambda j: (j, 0)),
        out_shape=jax.ShapeDtypeStruct((HALF, 2 * H), jnp.float32),
    )(table_t, table_t)


def _sc_mesh():
    return plsc.VectorSubcoreMesh(core_axis_name="c", subcore_axis_name="s")


def _sc_gather_items(pair_t, t2i):
    mesh = _sc_mesh()
    nc = mesh.num_cores

    @functools.partial(
        pl.kernel,
        out_type=jax.ShapeDtypeStruct((K, B, 128), jnp.float32),
        mesh=mesh,
        scratch_types=[
            pltpu.VMEM((K, BPW), jnp.int32),
            pltpu.VMEM((BPW, 128), jnp.float32),
            pltpu.VMEM((BPW, 128), jnp.float32),
            pltpu.SemaphoreType.DMA,
            pltpu.SemaphoreType.DMA,
        ],
    )
    def k(pair_hbm, t2i_hbm, pairs_out, pv, ra, rb, sema, semb):
        wid = lax.axis_index("s") * nc + lax.axis_index("c")
        b0 = wid * BPW

        pltpu.sync_copy(pair_hbm.at[:, pl.ds(b0, BPW)], pv)

        def gather(k_slot, buf, sem):
            pltpu.make_async_copy(t2i_hbm.at[pv.at[k_slot]], buf, sem).start()

        def flush(k_slot, buf, sem):
            pltpu.make_async_copy(t2i_hbm.at[pv.at[k_slot]], buf, sem).wait()
            pltpu.sync_copy(buf, pairs_out.at[k_slot, pl.ds(b0, BPW), :])

        gather(0, ra, sema)

        def body(j, _):
            k0 = 2 * j
            gather(k0 + 1, rb, semb)
            flush(k0, ra, sema)

            @pl.when(k0 + 2 < K)
            def _():
                gather(k0 + 2, ra, sema)

            flush(k0 + 1, rb, semb)
            return 0

        lax.fori_loop(0, K // 2, body, 0)

    return k(pair_t, t2i)


def _sc_gather_users(upair, t2u):
    mesh = _sc_mesh()
    nc = mesh.num_cores

    @functools.partial(
        pl.kernel,
        out_type=jax.ShapeDtypeStruct((B, 128), jnp.float32),
        mesh=mesh,
        scratch_types=[
            pltpu.VMEM((BPW,), jnp.int32),
            pltpu.VMEM((BPW, 128), jnp.float32),
            pltpu.SemaphoreType.DMA,
        ],
    )
    def k(upair_hbm, t2u_hbm, upairs_out, upv, ra, sema):
        wid = lax.axis_index("s") * nc + lax.axis_index("c")
        b0 = wid * BPW
        pltpu.sync_copy(upair_hbm.at[pl.ds(b0, BPW)], upv)
        pltpu.make_async_copy(t2u_hbm.at[upv], ra, sema).start()
        pltpu.make_async_copy(t2u_hbm.at[upv], ra, sema).wait()
        pltpu.sync_copy(ra, upairs_out.at[pl.ds(b0, BPW)])

    return k(upair, t2u)


def _tc_finish_body(idx_ref, users_ref, pairs_ref, upairs_ref,
                    uet_ref, allt_ref, rat_ref):
    up = upairs_ref[...]
    upt = up.T
    upar = users_ref[...] >= HALF
    ue = jnp.where(upar, upt[H:], upt[:H])
    uet_ref[...] = ue

    x = pairs_ref[...]
    xt = jnp.transpose(x, (0, 2, 1))
    par = (idx_ref[...] >= HALF)[:, None, :]
    sel = jnp.where(par, xt[:, H:, :], xt[:, :H, :])
    allt_ref[...] = sel
    rat_ref[...] = jnp.sum(sel * ue[None], axis=1)


def _tc_finish(idx_t, users2d, pairs, upairs):
    bb = 256
    return pl.pallas_call(
        _tc_finish_body,
        grid=(B // bb,),
        in_specs=[
            pl.BlockSpec((K, bb), lambda i: (0, i)),
            pl.BlockSpec((1, bb), lambda i: (0, i)),
            pl.BlockSpec((K, bb, 128), lambda i: (0, i, 0)),
            pl.BlockSpec((bb, 128), lambda i: (i, 0)),
        ],
        out_specs=[
            pl.BlockSpec((H, bb), lambda i: (0, i)),
            pl.BlockSpec((K, H, bb), lambda i: (0, 0, i)),
            pl.BlockSpec((K, bb), lambda i: (0, i)),
        ],
        out_shape=[
            jax.ShapeDtypeStruct((H, B), jnp.float32),
            jax.ShapeDtypeStruct((K, H, B), jnp.float32),
            jax.ShapeDtypeStruct((K, B), jnp.float32),
        ],
    )(idx_t, users2d, pairs, upairs)


def kernel(users, pos_items, neg_items, user_table, item_table):
    users = users.astype(jnp.int32)
    pos_items = pos_items.astype(jnp.int32)
    item_idx = jnp.concatenate(
        [pos_items[:, None], neg_items.astype(jnp.int32)], axis=1)
    idx_t = item_idx.T
    pair_t = idx_t & (HALF - 1)
    upair = users & (HALF - 1)

    t2i = _tc_build_pairs(item_table.T)
    t2u = _tc_build_pairs(user_table.T)

    pairs = _sc_gather_items(pair_t, t2i)
    upairs = _sc_gather_users(upair, t2u)
    ue_t, all_t, rating_t = _tc_finish(idx_t, users[None, :], pairs, upairs)
    pe_t = all_t[0]

    return (ue_t.T, pe_t.T, all_t.transpose(2, 0, 1), rating_t.T)
